# baseline (speedup 1.0000x reference)
import jax
import jax.numpy as jnp
from jax.experimental import pallas as pl
from jax.experimental.pallas import tpu as pltpu

_N = 10000
_IN_CH = 128
_OUT_CH = 64
_CAT = _IN_CH + _OUT_CH
_TILE = 1000


def _fused_tile(x_ref, wz_ref, bz_ref, wh_ref, bh_ref, fc1_ref, fc1b_ref,
                out_ref):
    i = pl.program_id(0)
    X = x_ref[...]
    nrm = jnp.sqrt(jnp.sum(X * X, axis=1, keepdims=True))
    Xn = X / jnp.maximum(nrm, 1e-12)
    Z = jax.nn.sigmoid(
        jnp.dot(Xn, wz_ref[0, :_IN_CH, :])
        + jnp.dot(Xn, wz_ref[1, :_IN_CH, :]) + bz_ref[...])
    T = jnp.tanh(
        jnp.dot(Xn, wh_ref[0, :_IN_CH, :])
        + jnp.dot(Xn, wh_ref[1, :_IN_CH, :]) + bh_ref[...])
    H = jnp.maximum((1.0 - Z) * T, 0.0)
    partial = jnp.sum(fc1_ref[...] * H).reshape(1, 1)

    @pl.when(i == 0)
    def _init():
        out_ref[...] = fc1b_ref[...]

    out_ref[...] += partial


def kernel(x, edge_index, edge_weight, Wz, bz, Wr, br, Wh, bh, fc1_w, fc1_b):
    del edge_index, edge_weight, Wr, br
    wz = Wz.reshape(2, _CAT, _OUT_CH)
    wh = Wh.reshape(2, _CAT, _OUT_CH)
    fc1_m = fc1_w.reshape(_N, _OUT_CH)
    grid = _N // _TILE
    out = pl.pallas_call(
        _fused_tile,
        grid=(grid,),
        in_specs=[
            pl.BlockSpec((_TILE, _IN_CH), lambda i: (i, 0)),
            pl.BlockSpec((2, _CAT, _OUT_CH), lambda i: (0, 0, 0)),
            pl.BlockSpec((1, _OUT_CH), lambda i: (0, 0)),
            pl.BlockSpec((2, _CAT, _OUT_CH), lambda i: (0, 0, 0)),
            pl.BlockSpec((1, _OUT_CH), lambda i: (0, 0)),
            pl.BlockSpec((_TILE, _OUT_CH), lambda i: (i, 0)),
            pl.BlockSpec((1, 1), lambda i: (0, 0)),
        ],
        out_specs=pl.BlockSpec((1, 1), lambda i: (0, 0)),
        out_shape=jax.ShapeDtypeStruct((1, 1), jnp.float32),
        compiler_params=pltpu.CompilerParams(
            dimension_semantics=("arbitrary",)),
    )(x, wz, bz.reshape(1, _OUT_CH), wh, bh.reshape(1, _OUT_CH), fc1_m,
      fc1_b.reshape(1, 1))
    return out.reshape(1)


# speedup vs baseline: 1.0919x; 1.0919x over previous
import jax
import jax.numpy as jnp
from jax.experimental import pallas as pl
from jax.experimental.pallas import tpu as pltpu

_N = 10000
_IN_CH = 128
_OUT_CH = 64
_CAT = _IN_CH + _OUT_CH
_TILE = 2000


def _fused_tile(x_ref, wz_ref, bz_ref, wh_ref, bh_ref, fc1_ref, fc1b_ref,
                out_ref):
    i = pl.program_id(0)
    X = x_ref[...]
    nrm = jnp.sqrt(jnp.sum(X * X, axis=1, keepdims=True))
    Xn = X / jnp.maximum(nrm, 1e-12)
    Z = jax.nn.sigmoid(
        jnp.dot(Xn, wz_ref[0, :_IN_CH, :])
        + jnp.dot(Xn, wz_ref[1, :_IN_CH, :]) + bz_ref[...])
    T = jnp.tanh(
        jnp.dot(Xn, wh_ref[0, :_IN_CH, :])
        + jnp.dot(Xn, wh_ref[1, :_IN_CH, :]) + bh_ref[...])
    H = jnp.maximum((1.0 - Z) * T, 0.0)
    partial = jnp.sum(fc1_ref[...] * H).reshape(1, 1)

    @pl.when(i == 0)
    def _init():
        out_ref[...] = fc1b_ref[...]

    out_ref[...] += partial


def kernel(x, edge_index, edge_weight, Wz, bz, Wr, br, Wh, bh, fc1_w, fc1_b):
    del edge_index, edge_weight, Wr, br
    wz = Wz.reshape(2, _CAT, _OUT_CH)
    wh = Wh.reshape(2, _CAT, _OUT_CH)
    fc1_m = fc1_w.reshape(_N, _OUT_CH)
    grid = _N // _TILE
    out = pl.pallas_call(
        _fused_tile,
        grid=(grid,),
        in_specs=[
            pl.BlockSpec((_TILE, _IN_CH), lambda i: (i, 0)),
            pl.BlockSpec((2, _CAT, _OUT_CH), lambda i: (0, 0, 0)),
            pl.BlockSpec((1, _OUT_CH), lambda i: (0, 0)),
            pl.BlockSpec((2, _CAT, _OUT_CH), lambda i: (0, 0, 0)),
            pl.BlockSpec((1, _OUT_CH), lambda i: (0, 0)),
            pl.BlockSpec((_TILE, _OUT_CH), lambda i: (i, 0)),
            pl.BlockSpec((1, 1), lambda i: (0, 0)),
        ],
        out_specs=pl.BlockSpec((1, 1), lambda i: (0, 0)),
        out_shape=jax.ShapeDtypeStruct((1, 1), jnp.float32),
        compiler_params=pltpu.CompilerParams(
            dimension_semantics=("arbitrary",)),
    )(x, wz, bz.reshape(1, _OUT_CH), wh, bh.reshape(1, _OUT_CH), fc1_m,
      fc1_b.reshape(1, 1))
    return out.reshape(1)
